# Initial kernel scaffold; baseline (speedup 1.0000x reference)
import functools

import jax
import jax.numpy as jnp
from jax import lax
from jax.experimental import pallas as pl
from jax.experimental.pallas import tpu as pltpu
from jax.experimental.pallas import tpu_sc as plsc

VOCAB = 100000
EMB = 128
B = 4096
L = 200

NC = 2
NS = 16
NW = NC * NS
BPW = B // NW
LANES = 16
NCHUNK = EMB // LANES
HALF = L // 2


def _pool_body(src_hbm, table_hbm, out_hbm, idx_all, rows0, rows1,
               out_stage, sem0, sem1):
    wid = lax.axis_index("s") * NC + lax.axis_index("c")
    base = wid * BPW

    pltpu.sync_copy(src_hbm.at[pl.ds(base, BPW)], idx_all)

    rows = (rows0, rows1)
    sems = (sem0, sem1)

    def issue(i, buf):
        for j in range(2):
            pltpu.async_copy(
                table_hbm.at[idx_all.at[i, pl.ds(j * HALF, HALF)]],
                rows[buf].at[pl.ds(j * HALF, HALF)],
                sems[buf],
            )

    def drain(i, buf):
        for j in range(2):
            pltpu.make_async_copy(
                table_hbm.at[idx_all.at[i, pl.ds(j * HALF, HALF)]],
                rows[buf].at[pl.ds(j * HALF, HALF)],
                sems[buf],
            ).wait()

    issue(0, 0)
    issue(1, 1)

    def step(i0, carry):
        for buf in range(2):
            i = i0 * 2 + buf
            drain(i, buf)

            def body(r, acc):
                row = rows[buf].at[r]
                return tuple(
                    acc[c] + row[pl.ds(c * LANES, LANES)]
                    for c in range(NCHUNK)
                )

            zeros = tuple(
                jnp.zeros((LANES,), jnp.float32) for _ in range(NCHUNK))
            acc = lax.fori_loop(0, L, body, zeros)
            for c in range(NCHUNK):
                out_stage[i, pl.ds(c * LANES, LANES)] = acc[c]

            @pl.when(i + 2 < BPW)
            def _():
                issue(i + 2, buf)
        return carry

    lax.fori_loop(0, BPW // 2, step, 0)

    pltpu.sync_copy(out_stage, out_hbm.at[pl.ds(base, BPW)])


def _sc_pool(src32, table):
    mesh = plsc.VectorSubcoreMesh(core_axis_name="c", subcore_axis_name="s")
    f = pl.kernel(
        _pool_body,
        out_type=jax.ShapeDtypeStruct((B, EMB), jnp.float32),
        mesh=mesh,
        scratch_types=[
            pltpu.VMEM((BPW, L), jnp.int32),
            pltpu.VMEM((L, EMB), jnp.float32),
            pltpu.VMEM((L, EMB), jnp.float32),
            pltpu.VMEM((BPW, EMB), jnp.float32),
            pltpu.SemaphoreType.DMA,
            pltpu.SemaphoreType.DMA,
        ],
    )
    return f(src32, table)


def _ffn_body(x_ref, w_ref, b_ref, o_ref):
    x = x_ref[...] * (1.0 / L)
    o_ref[...] = lax.dot_general(
        x, w_ref[...], (((1,), (1,)), ((), ())),
        preferred_element_type=jnp.float32) + b_ref[...]


def _tc_ffn(sums, W, b):
    blk = 512
    grid = (B // blk,)
    return pl.pallas_call(
        _ffn_body,
        grid=grid,
        in_specs=[
            pl.BlockSpec((blk, EMB), lambda i: (i, 0)),
            pl.BlockSpec((EMB, EMB), lambda i: (0, 0)),
            pl.BlockSpec((1, EMB), lambda i: (0, 0)),
        ],
        out_specs=pl.BlockSpec((blk, EMB), lambda i: (i, 0)),
        out_shape=jax.ShapeDtypeStruct((B, EMB), jnp.float32),
    )(sums, W, b.reshape(1, EMB))


@jax.jit
def kernel(src, table, W, b):
    src32 = src.astype(jnp.int32)
    sums = _sc_pool(src32, table)
    hidden = _tc_ffn(sums, W, b)
    return hidden[None, :, :]


# speedup vs baseline: 13.0161x; 13.0161x over previous
import functools

import jax
import jax.numpy as jnp
from jax import lax
from jax.experimental import pallas as pl
from jax.experimental.pallas import tpu as pltpu
from jax.experimental.pallas import tpu_sc as plsc

VOCAB = 100000
EMB = 128
B = 4096
L = 200

NC = 2
NS = 16
NW = NC * NS
BPW = B // NW
LANES = 16
NCHUNK = EMB // LANES
SPLITS = ((0, 104), (104, 96))


def _pool_body(src_hbm, table_hbm, out_hbm, idx_all, rows0, rows1,
               out_stage, sem0, sem1):
    wid = lax.axis_index("s") * NC + lax.axis_index("c")
    base = wid * BPW

    pltpu.sync_copy(src_hbm.at[pl.ds(base * L, BPW * L)], idx_all)

    rows = (rows0, rows1)
    sems = (sem0, sem1)

    def issue(i, buf):
        for off, n in SPLITS:
            pltpu.async_copy(
                table_hbm.at[idx_all.at[pl.ds(i * L + off, n)]],
                rows[buf].at[pl.ds(off, n)],
                sems[buf],
            )

    def drain(i, buf):
        for off, n in SPLITS:
            pltpu.make_async_copy(
                table_hbm.at[idx_all.at[pl.ds(i * L + off, n)]],
                rows[buf].at[pl.ds(off, n)],
                sems[buf],
            ).wait()

    issue(0, 0)
    issue(1, 1)

    def step(i0, carry):
        for buf in range(2):
            i = i0 * 2 + buf
            drain(i, buf)

            def body(r, acc):
                row = rows[buf].at[r]
                return tuple(
                    acc[c] + row[pl.ds(c * LANES, LANES)]
                    for c in range(NCHUNK)
                )

            zeros = tuple(
                jnp.zeros((LANES,), jnp.float32) for _ in range(NCHUNK))
            acc = lax.fori_loop(0, L, body, zeros)
            for c in range(NCHUNK):
                out_stage[i, pl.ds(c * LANES, LANES)] = acc[c]

            @pl.when(i + 2 < BPW)
            def _():
                issue(i + 2, buf)
        return carry

    lax.fori_loop(0, BPW // 2, step, 0)

    pltpu.sync_copy(out_stage, out_hbm.at[pl.ds(base, BPW)])


def _sc_pool(src32, table):
    mesh = plsc.VectorSubcoreMesh(core_axis_name="c", subcore_axis_name="s")
    f = pl.kernel(
        _pool_body,
        out_type=jax.ShapeDtypeStruct((B, EMB), jnp.float32),
        mesh=mesh,
        scratch_types=[
            pltpu.VMEM((BPW * L,), jnp.int32),
            pltpu.VMEM((L, EMB), jnp.float32),
            pltpu.VMEM((L, EMB), jnp.float32),
            pltpu.VMEM((BPW, EMB), jnp.float32),
            pltpu.SemaphoreType.DMA,
            pltpu.SemaphoreType.DMA,
        ],
    )
    return f(src32, table)


def _ffn_body(x_ref, w_ref, b_ref, o_ref):
    x = x_ref[...] * (1.0 / L)
    o_ref[...] = lax.dot_general(
        x, w_ref[...], (((1,), (1,)), ((), ())),
        preferred_element_type=jnp.float32) + b_ref[...]


def _tc_ffn(sums, W, b):
    blk = 512
    grid = (B // blk,)
    return pl.pallas_call(
        _ffn_body,
        grid=grid,
        in_specs=[
            pl.BlockSpec((blk, EMB), lambda i: (i, 0)),
            pl.BlockSpec((EMB, EMB), lambda i: (0, 0)),
            pl.BlockSpec((1, EMB), lambda i: (0, 0)),
        ],
        out_specs=pl.BlockSpec((blk, EMB), lambda i: (i, 0)),
        out_shape=jax.ShapeDtypeStruct((B, EMB), jnp.float32),
    )(sums, W, b.reshape(1, EMB))


@jax.jit
def kernel(src, table, W, b):
    src32 = src.astype(jnp.int32).reshape(B * L)
    sums = _sc_pool(src32, table)
    hidden = _tc_ffn(sums, W, b)
    return hidden[None, :, :]


# speedup vs baseline: 13.0775x; 1.0047x over previous
import functools

import jax
import jax.numpy as jnp
from jax import lax
from jax.experimental import pallas as pl
from jax.experimental.pallas import tpu as pltpu
from jax.experimental.pallas import tpu_sc as plsc

VOCAB = 100000
EMB = 128
B = 4096
L = 200

NC = 2
NS = 16
NW = NC * NS
BPW = B // NW
LANES = 16
NCHUNK = EMB // LANES
SPLITS = ((0, 104), (104, 96))


def _pool_body(src_hbm, table_hbm, out_hbm, idx_all, rows0, rows1,
               out_stage, sem0, sem1):
    wid = lax.axis_index("s") * NC + lax.axis_index("c")
    base = wid * BPW

    pltpu.sync_copy(src_hbm.at[pl.ds(base * L, BPW * L)], idx_all)

    rows = (rows0, rows1)
    sems = (sem0, sem1)

    def issue(i, buf):
        for off, n in SPLITS:
            pltpu.async_copy(
                table_hbm.at[idx_all.at[pl.ds(i * L + off, n)]],
                rows[buf].at[pl.ds(off, n)],
                sems[buf],
            )

    def drain(i, buf):
        for off, n in SPLITS:
            pltpu.make_async_copy(
                table_hbm.at[idx_all.at[pl.ds(i * L + off, n)]],
                rows[buf].at[pl.ds(off, n)],
                sems[buf],
            ).wait()

    issue(0, 0)
    issue(1, 1)

    def step(i0, carry):
        for buf in range(2):
            i = i0 * 2 + buf
            drain(i, buf)

            def body(r, acc):
                row = rows[buf].at[r]
                return tuple(
                    acc[c] + row[pl.ds(c * LANES, LANES)]
                    for c in range(NCHUNK)
                )

            zeros = tuple(
                jnp.zeros((LANES,), jnp.float32) for _ in range(NCHUNK))
            acc = lax.fori_loop(0, L, body, zeros, unroll=4)
            for c in range(NCHUNK):
                out_stage[i, pl.ds(c * LANES, LANES)] = acc[c]

            @pl.when(i + 2 < BPW)
            def _():
                issue(i + 2, buf)
        return carry

    lax.fori_loop(0, BPW // 2, step, 0)

    pltpu.sync_copy(out_stage, out_hbm.at[pl.ds(base, BPW)])


def _sc_pool(src32, table):
    mesh = plsc.VectorSubcoreMesh(core_axis_name="c", subcore_axis_name="s")
    f = pl.kernel(
        _pool_body,
        out_type=jax.ShapeDtypeStruct((B, EMB), jnp.float32),
        mesh=mesh,
        scratch_types=[
            pltpu.VMEM((BPW * L,), jnp.int32),
            pltpu.VMEM((L, EMB), jnp.float32),
            pltpu.VMEM((L, EMB), jnp.float32),
            pltpu.VMEM((BPW, EMB), jnp.float32),
            pltpu.SemaphoreType.DMA,
            pltpu.SemaphoreType.DMA,
        ],
    )
    return f(src32, table)


def _ffn_body(x_ref, w_ref, b_ref, o_ref):
    x = x_ref[...] * (1.0 / L)
    o_ref[...] = lax.dot_general(
        x, w_ref[...], (((1,), (1,)), ((), ())),
        preferred_element_type=jnp.float32) + b_ref[...]


def _tc_ffn(sums, W, b):
    blk = 512
    grid = (B // blk,)
    return pl.pallas_call(
        _ffn_body,
        grid=grid,
        in_specs=[
            pl.BlockSpec((blk, EMB), lambda i: (i, 0)),
            pl.BlockSpec((EMB, EMB), lambda i: (0, 0)),
            pl.BlockSpec((1, EMB), lambda i: (0, 0)),
        ],
        out_specs=pl.BlockSpec((blk, EMB), lambda i: (i, 0)),
        out_shape=jax.ShapeDtypeStruct((B, EMB), jnp.float32),
    )(sums, W, b.reshape(1, EMB))


@jax.jit
def kernel(src, table, W, b):
    src32 = src.astype(jnp.int32).reshape(B * L)
    sums = _sc_pool(src32, table)
    hidden = _tc_ffn(sums, W, b)
    return hidden[None, :, :]
